# baseline (speedup 1.0000x reference)
import jax
import jax.numpy as jnp
from jax import lax
from jax.experimental import pallas as pl
from jax.experimental.pallas import tpu as pltpu
from jax.experimental.pallas import tpu_sc as plsc

R = 32
A = 8
N = 4096
EPS = 1e-12
NC = 2
NS = 16
L = 16
NW = NC * NS
SPW = N // NW
NB = SPW // L
PF = 264
PAD = 8


def _bcast(ref, i):
    return plsc.load_gather(ref, [jnp.full((L,), i, jnp.int32)])


def _sc_body(xt_hbm, frb_hbm, c1_hbm, c2_hbm, out_hbm,
             x_v, f_v, c1_v, c2_v, m_v, wu_v, wl_v,
             b2_v, b1_v, rk1_v, rk2_v, d1_v, d2_v, out_v):
    wid = lax.axis_index("s") * NC + lax.axis_index("c")
    base = wid * SPW
    pltpu.sync_copy(xt_hbm.at[:, pl.ds(base, SPW)], x_v)
    pltpu.sync_copy(frb_hbm.at[pl.ds(0, PF)], f_v)
    pltpu.sync_copy(c1_hbm, c1_v.at[pl.ds(PAD, R)])
    pltpu.sync_copy(c2_hbm, c2_v.at[pl.ds(PAD, R)])
    iota = lax.iota(jnp.int32, L)

    for chunk in range(R * A // L):
        b0 = chunk * L
        mv = f_v[pl.ds(b0, L)]
        sav = plsc.load_gather(f_v, [iota + (b0 + 1)])
        sbv = plsc.load_gather(f_v, [iota + (b0 + 2)])
        sbig = jnp.maximum(sav, sbv)
        ssm = jnp.minimum(sav, sbv)
        wuv = -0.5 / (sbig * sbig)
        wlv = -0.5 / (ssm * ssm)
        for q in range(L):
            p = b0 + q
            m_v[pl.ds(p * L, L)] = jnp.full((L,), mv[q])
            wu_v[pl.ds(p * L, L)] = jnp.full((L,), wuv[q])
            wl_v[pl.ds(p * L, L)] = jnp.full((L,), wlv[q])

    for c_v, b_v, rk_v in ((c1_v, b2_v, rk1_v), (c2_v, b1_v, rk2_v)):
        ci0 = c_v[pl.ds(PAD, L)]
        ci1 = c_v[pl.ds(PAD + L, L)]
        cnt0 = jnp.zeros((L,), jnp.int32)
        cnt1 = jnp.zeros((L,), jnp.int32)
        for j in range(R):
            cj = (ci0, ci1)[j // L][j % L]
            win0 = (cj < ci0) | ((cj == ci0) & (j < iota))
            win1 = (cj < ci1) | ((cj == ci1) & (j < iota + L))
            cnt0 = cnt0 + jnp.where(win0, 1, 0)
            cnt1 = cnt1 + jnp.where(win1, 1, 0)
        rk_v[pl.ds(0, L)] = cnt0
        rk_v[pl.ds(L, L)] = cnt1
        plsc.store_scatter(b_v, [cnt0], ci0)
        plsc.store_scatter(b_v, [cnt1], ci1)

    def blk_body(blk, carry):
        col = blk * L + iota
        xs = [plsc.load_gather(x_v, [jnp.full((L,), a, jnp.int32), col])
              for a in range(A)]
        c1c = (c1_v[pl.ds(PAD, L)], c1_v[pl.ds(PAD + L, L)])
        c2c = (c2_v[pl.ds(PAD, L)], c2_v[pl.ds(PAD + L, L)])
        rk1c = (rk1_v[pl.ds(0, L)], rk1_v[pl.ds(L, L)])
        rk2c = (rk2_v[pl.ds(0, L)], rk2_v[pl.ds(L, L)])

        s0l = jnp.zeros((L,), jnp.float32)
        t0l = jnp.zeros((L,), jnp.float32)
        s0r = jnp.zeros((L,), jnp.float32)
        t0r = jnp.zeros((L,), jnp.float32)
        for r in range(R):
            au = None
            al = None
            for a in range(A):
                p = r * A + a
                d = xs[a] - m_v[pl.ds(p * L, L)]
                d2 = d * d
                if a == 0:
                    au = d2 * wu_v[pl.ds(p * L, L)]
                    al = d2 * wl_v[pl.ds(p * L, L)]
                else:
                    au = au + d2 * wu_v[pl.ds(p * L, L)]
                    al = al + d2 * wl_v[pl.ds(p * L, L)]
            uu = jnp.exp(au)
            ll = jnp.exp(al)
            hi, lo = r // L, r % L
            s0l = s0l + c1c[hi][lo] * ll
            t0l = t0l + ll
            s0r = s0r + c2c[hi][lo] * uu
            t0r = t0r + uu
            dlt = uu - ll
            plsc.store_scatter(d1_v, [rk1c[hi][lo] * L + iota], dlt)
            plsc.store_scatter(d2_v, [rk2c[hi][lo] * L + iota], dlt)

        b2c = (b2_v[pl.ds(0, L)], b2_v[pl.ds(L, L)])
        s = s0l
        t = t0l + EPS
        sbl = s
        tbl = t
        for k in range(R):
            dk = d1_v[pl.ds(k * L, L)]
            s = s + b2c[k // L][k % L] * dk
            t = t + dk
            cond = s * tbl < sbl * t
            sbl = jnp.where(cond, s, sbl)
            tbl = jnp.where(cond, t, tbl)

        b1c = (b1_v[pl.ds(0, L)], b1_v[pl.ds(L, L)])
        s = s0r
        t = t0r + EPS
        sbr = s
        tbr = t
        for k in range(R):
            dk = d2_v[pl.ds(k * L, L)]
            s = s - b1c[k // L][k % L] * dk
            t = t - dk
            cond = s * tbr > sbr * t
            sbr = jnp.where(cond, s, sbr)
            tbr = jnp.where(cond, t, tbr)

        plsc.store_scatter(out_v, [col], (sbl / tbl + sbr / tbr) * 0.5)
        return carry

    lax.fori_loop(0, NB, blk_body, 0)
    pltpu.sync_copy(out_v, out_hbm.at[pl.ds(base, SPW)])


_km_kernel = pl.kernel(
    _sc_body,
    out_type=jax.ShapeDtypeStruct((N,), jnp.float32),
    mesh=plsc.VectorSubcoreMesh(
        core_axis_name="c", subcore_axis_name="s",
        num_cores=NC, num_subcores=NS),
    compiler_params=pltpu.CompilerParams(needs_layout_passes=False),
    scratch_types=[
        pltpu.VMEM((A, SPW), jnp.float32),
        pltpu.VMEM((PF,), jnp.float32),
        pltpu.VMEM((R + PAD,), jnp.float32),
        pltpu.VMEM((R + PAD,), jnp.float32),
        pltpu.VMEM((R * A * L,), jnp.float32),
        pltpu.VMEM((R * A * L,), jnp.float32),
        pltpu.VMEM((R * A * L,), jnp.float32),
        pltpu.VMEM((R,), jnp.float32),
        pltpu.VMEM((R,), jnp.float32),
        pltpu.VMEM((R,), jnp.int32),
        pltpu.VMEM((R,), jnp.int32),
        pltpu.VMEM((R * L,), jnp.float32),
        pltpu.VMEM((R * L,), jnp.float32),
        pltpu.VMEM((SPW,), jnp.float32),
    ],
)


@jax.jit
def kernel(input_data, FRB_weights, c1, c2):
    return _km_kernel(input_data.T, FRB_weights, c1, c2)


# speedup vs baseline: 8.6858x; 8.6858x over previous
import jax
import jax.numpy as jnp
from jax import lax
from jax.experimental import pallas as pl

R = 32
A = 8
N = 4096
EPS = 1e-12


def _km_body(xt_ref, m_ref, sa_ref, sb_ref, c1_ref, c2_ref, out_ref):
    xt = xt_ref[...]
    m = m_ref[...]
    sa = sa_ref[...]
    sb = sb_ref[...]
    c1r = c1_ref[...]
    c2r = c2_ref[...]

    sbig = jnp.maximum(sa, sb)
    ssml = jnp.minimum(sa, sb)
    wu = -0.5 / (sbig * sbig)
    wl = -0.5 / (ssml * ssml)

    x2t = xt * xt
    f32 = jnp.float32
    ku = jnp.sum(wu * m * m, axis=1, keepdims=True)
    kl = jnp.sum(wl * m * m, axis=1, keepdims=True)
    au = (jnp.dot(wu, x2t, preferred_element_type=f32)
          + jnp.dot(-2.0 * m * wu, xt, preferred_element_type=f32) + ku)
    al = (jnp.dot(wl, x2t, preferred_element_type=f32)
          + jnp.dot(-2.0 * m * wl, xt, preferred_element_type=f32) + kl)
    uu = jnp.exp(au)
    ll = jnp.exp(al)
    dlt = uu - ll

    io = lax.broadcasted_iota(jnp.int32, (R, R), 0)
    ii = lax.broadcasted_iota(jnp.int32, (R, R), 1)
    eye = (io == ii).astype(f32)
    ones = jnp.ones((R, R), f32)

    def rank_row(cr):
        ccol = jnp.dot(eye * cr, ones, preferred_element_type=f32)
        win = (ccol < cr) | ((ccol == cr) & (io < ii))
        return jnp.sum(win.astype(jnp.int32), axis=0, keepdims=True)

    rk1 = rank_row(c1r)
    rk2 = rank_row(c2r)
    m1 = (io >= rk1).astype(f32)
    m2 = (io >= rk2).astype(f32)

    c1col = jnp.dot(eye * c1r, ones, preferred_element_type=f32)[:, 0:1]
    c2col = jnp.dot(eye * c2r, ones, preferred_element_type=f32)[:, 0:1]
    s0l = jnp.sum(c1col * ll, axis=0, keepdims=True)
    t0l = jnp.sum(ll, axis=0, keepdims=True)
    s0r = jnp.sum(c2col * uu, axis=0, keepdims=True)
    t0r = jnp.sum(uu, axis=0, keepdims=True)

    s_cum = jnp.dot(m1 * c1r, dlt, preferred_element_type=f32)
    t_cum = jnp.dot(m1, dlt, preferred_element_type=f32)
    ratl = (s0l + s_cum) / (t0l + t_cum + EPS)
    lmin = jnp.minimum(jnp.min(ratl, axis=0, keepdims=True),
                       s0l / (t0l + EPS))

    s_cum2 = jnp.dot(m2 * c2r, dlt, preferred_element_type=f32)
    t_cum2 = jnp.dot(m2, dlt, preferred_element_type=f32)
    ratr = (s0r - s_cum2) / (t0r - t_cum2 + EPS)
    rmax = jnp.maximum(jnp.max(ratr, axis=0, keepdims=True),
                       s0r / (t0r + EPS))

    out_ref[...] = (lmin + rmax) * 0.5


_km_call = pl.pallas_call(
    _km_body,
    out_shape=jax.ShapeDtypeStruct((1, N), jnp.float32),
)


@jax.jit
def kernel(input_data, FRB_weights, c1, c2):
    xt = input_data.T
    m = FRB_weights[0:R * A].reshape(R, A)
    sa = FRB_weights[1:R * A + 1].reshape(R, A)
    sb = FRB_weights[2:R * A + 2].reshape(R, A)
    y = _km_call(xt, m, sa, sb, c1.reshape(1, R), c2.reshape(1, R))
    return y.reshape(N)
